# Initial kernel scaffold; baseline (speedup 1.0000x reference)
import jax
import jax.numpy as jnp
from jax.experimental import pallas as pl


def kernel(x, edge_index, edge_attr, batch, embed_table, w_ih_f, w_hh_f, b_ih_f, b_hh_f, w_ih_b, w_hh_b, b_ih_b, b_hh_b, pw1, pb1, pw2, pb2, pw3, pb3, pw4, pb4, vw1, vb1, vw2, vb2):
    raise NotImplementedError("write your pallas kernel here")


# speedup vs baseline: 5.0476x; 5.0476x over previous
import functools

import jax
import jax.numpy as jnp
from jax import lax
from jax.experimental import pallas as pl
from jax.experimental.pallas import tpu as pltpu
from jax.experimental.pallas import tpu_sc as plsc

MAX_LEN = 20
SND = 53
VOCAB_PAD = 32
H = 32
DROW = 128
CNT_COL = SND + 2 * H
ACTION_SIZE = 53

EBLK = 1000
GBLK = 40


def _lstm_body(ea_ref, w_ref, b_ref, out_ref, in_ref):
    f32 = jnp.float32
    ea = ea_ref[...]
    B = ea.shape[0]
    W = w_ref[...]
    bias = b_ref[...]
    iota = lax.broadcasted_iota(jnp.int32, (B, VOCAB_PAD), 1)
    h_f = jnp.zeros((B, H), f32)
    c_f = jnp.zeros((B, H), f32)
    h_b = jnp.zeros((B, H), f32)
    c_b = jnp.zeros((B, H), f32)
    acc = jnp.zeros((B, 2 * H), f32)
    for t in range(MAX_LEN):
        onef = (ea[:, t:t + 1] == iota).astype(f32)
        oneb = (ea[:, MAX_LEN - 1 - t:MAX_LEN - t] == iota).astype(f32)
        inp = jnp.concatenate([onef, h_f, oneb, h_b], axis=1)
        g = jnp.dot(inp, W, preferred_element_type=f32) + bias
        i_f = jax.nn.sigmoid(g[:, 0:32])
        f_f = jax.nn.sigmoid(g[:, 32:64])
        g_f = jnp.tanh(g[:, 64:96])
        o_f = jax.nn.sigmoid(g[:, 96:128])
        c_f = f_f * c_f + i_f * g_f
        h_f = o_f * jnp.tanh(c_f)
        i_b = jax.nn.sigmoid(g[:, 128:160])
        f_b = jax.nn.sigmoid(g[:, 160:192])
        g_b = jnp.tanh(g[:, 192:224])
        o_b = jax.nn.sigmoid(g[:, 224:256])
        c_b = f_b * c_b + i_b * g_b
        h_b = o_b * jnp.tanh(c_b)
        acc = acc + jnp.concatenate([h_f, h_b], axis=1)
    regex = acc * (1.0 / MAX_LEN)
    src_sn = ea[:, MAX_LEN:MAX_LEN + SND].astype(f32)
    tgt_sn = ea[:, MAX_LEN + SND:MAX_LEN + 2 * SND].astype(f32)
    ones = jnp.ones((B, 1), f32)
    pad = jnp.zeros((B, DROW - CNT_COL - 1), f32)
    out_ref[...] = jnp.concatenate([tgt_sn, regex, ones, pad], axis=1)
    in_ref[...] = jnp.concatenate([src_sn, regex, ones, pad], axis=1)


def _run_stage1(edge_attr, W, bias, interpret=False):
    E, A = edge_attr.shape
    grid = (E // EBLK,)
    return pl.pallas_call(
        _lstm_body,
        grid=grid,
        in_specs=[
            pl.BlockSpec((EBLK, A), lambda i: (i, 0)),
            pl.BlockSpec((128, 256), lambda i: (0, 0)),
            pl.BlockSpec((1, 256), lambda i: (0, 0)),
        ],
        out_specs=[
            pl.BlockSpec((EBLK, DROW), lambda i: (i, 0)),
            pl.BlockSpec((EBLK, DROW), lambda i: (i, 0)),
        ],
        out_shape=[
            jax.ShapeDtypeStruct((E, DROW), jnp.float32),
            jax.ShapeDtypeStruct((E, DROW), jnp.float32),
        ],
        interpret=interpret,
    )(edge_attr, W, bias)


def _run_stage2(data_out, data_in, src, tgt, zeros_hbm, n_nodes):
    E = src.shape[0]
    NS = 16
    ET = E // NS
    CH = 128
    NFULL = ET // CH
    TAIL = ET - NFULL * CH
    TR = (n_nodes // NS) // 8 * 8
    TREM = n_nodes - TR * NS

    mesh = plsc.VectorSubcoreMesh(core_axis_name="c", subcore_axis_name="s")

    scratch = [
        pltpu.VMEM((CH, DROW), jnp.float32),
        pltpu.VMEM((CH,), jnp.int32),
        pltpu.VMEM_SHARED((n_nodes, DROW), jnp.float32),
    ]
    if TAIL:
        scratch += [
            pltpu.VMEM((TAIL, DROW), jnp.float32),
            pltpu.VMEM((TAIL,), jnp.int32),
        ]

    @functools.partial(
        pl.kernel,
        mesh=mesh,
        out_type=[
            jax.ShapeDtypeStruct((n_nodes, DROW), jnp.float32),
            jax.ShapeDtypeStruct((n_nodes, DROW), jnp.float32),
        ],
        scratch_types=scratch,
    )
    def sck(do_hbm, di_hbm, src_hbm, tgt_hbm, z_hbm, oo_hbm, oi_hbm,
            buf, idxv, acc, *tailbufs):
        c = lax.axis_index("c")
        s = lax.axis_index("s")
        pltpu.sync_copy(z_hbm.at[pl.ds(0, TR)], acc.at[pl.ds(s * TR, TR)])
        if TREM:
            @pl.when(s == NS - 1)
            def _():
                pltpu.sync_copy(z_hbm.at[pl.ds(0, TREM)],
                                acc.at[pl.ds(NS * TR, TREM)])
        plsc.subcore_barrier()

        def do_scatter(d_hbm, i_hbm):
            base = s * ET

            def body(k, carry):
                off = pl.multiple_of(base + k * CH, 8)
                pltpu.sync_copy(i_hbm.at[pl.ds(off, CH)], idxv)
                pltpu.sync_copy(d_hbm.at[pl.ds(off, CH)], buf)
                pltpu.sync_copy(buf, acc.at[idxv], add=True)
                return carry

            lax.fori_loop(0, NFULL, body, 0)
            if TAIL:
                tbuf, tidx = tailbufs
                off = base + NFULL * CH
                pltpu.sync_copy(i_hbm.at[pl.ds(off, TAIL)], tidx)
                pltpu.sync_copy(d_hbm.at[pl.ds(off, TAIL)], tbuf)
                pltpu.sync_copy(tbuf, acc.at[tidx], add=True)

        @pl.when(c == 0)
        def _():
            do_scatter(do_hbm, src_hbm)

        @pl.when(c == 1)
        def _():
            do_scatter(di_hbm, tgt_hbm)

        plsc.subcore_barrier()

        def writeout(o_hbm):
            pltpu.sync_copy(acc.at[pl.ds(s * TR, TR)], o_hbm.at[pl.ds(s * TR, TR)])
            if TREM:
                @pl.when(s == NS - 1)
                def _():
                    pltpu.sync_copy(acc.at[pl.ds(NS * TR, TREM)],
                                    o_hbm.at[pl.ds(NS * TR, TREM)])

        @pl.when(c == 0)
        def _():
            writeout(oo_hbm)

        @pl.when(c == 1)
        def _():
            writeout(oi_hbm)

    return sck(data_out, data_in, src, tgt, zeros_hbm)


def _mlp_body(x_ref, si_ref, so_ref,
              p1x_ref, p1i_ref, p1o_ref, pb1_ref,
              p2_ref, pb2_ref, p3_ref, pb3_ref, p4_ref, pb4_ref,
              v1x_ref, v1i_ref, v1o_ref, vb1_ref, v2_ref, vb2_ref,
              v_ref, pi_ref):
    f32 = jnp.float32
    x = x_ref[...]
    si = si_ref[...]
    so = so_ref[...]
    R = x.shape[0]
    cnt_i = jnp.maximum(si[:, CNT_COL:CNT_COL + 1], 1.0)
    cnt_o = jnp.maximum(so[:, CNT_COL:CNT_COL + 1], 1.0)
    inf = si / cnt_i
    outf = so / cnt_o

    def dot(a, b):
        return jnp.dot(a, b, preferred_element_type=f32)

    pi1 = jnp.maximum(dot(x, p1x_ref[...]) + dot(inf, p1i_ref[...])
                      + dot(outf, p1o_ref[...]) + pb1_ref[...], 0.0)
    pi2 = jnp.maximum(dot(pi1, p2_ref[...]) + pb2_ref[...], 0.0)
    pi3 = jnp.maximum(dot(pi2, p3_ref[...]) + pb3_ref[...], 0.0)
    pi4 = dot(pi3, p4_ref[...]) + pb4_ref[...]
    pi_ref[...] = pi4

    rows_per_graph = R // GBLK
    rg = lax.broadcasted_iota(jnp.int32, (GBLK, R), 1) // rows_per_graph
    gi = lax.broadcasted_iota(jnp.int32, (GBLK, R), 0)
    pool = (rg == gi).astype(f32) * (1.0 / rows_per_graph)
    mx = dot(pool, x)
    mi = dot(pool, inf)
    mo = dot(pool, outf)
    sv = jnp.maximum(dot(mx, v1x_ref[...]) + dot(mi, v1i_ref[...])
                     + dot(mo, v1o_ref[...]) + vb1_ref[...], 0.0)
    v_ref[...] = dot(sv, v2_ref[...]) + vb2_ref[...]


def _run_stage3(x, sums_in, sums_out, wts, n_graphs, interpret=False):
    N, XD = x.shape
    R = (N // n_graphs) * GBLK
    grid = (N // R,)

    def const_spec(a):
        return pl.BlockSpec(a.shape, lambda i: tuple(0 for _ in a.shape))

    in_specs = [
        pl.BlockSpec((R, XD), lambda i: (i, 0)),
        pl.BlockSpec((R, DROW), lambda i: (i, 0)),
        pl.BlockSpec((R, DROW), lambda i: (i, 0)),
    ] + [const_spec(w) for w in wts]
    return pl.pallas_call(
        _mlp_body,
        grid=grid,
        in_specs=in_specs,
        out_specs=[
            pl.BlockSpec((GBLK, 1), lambda i: (i, 0)),
            pl.BlockSpec((R, 1), lambda i: (i, 0)),
        ],
        out_shape=[
            jax.ShapeDtypeStruct((n_graphs, 1), jnp.float32),
            jax.ShapeDtypeStruct((N, 1), jnp.float32),
        ],
        interpret=interpret,
    )(x, sums_in, sums_out, *wts)


def _nx_body(pir_ref, nx_ref):
    p = pir_ref[...]
    Gb = p.shape[0]
    neg = jnp.full((Gb, ACTION_SIZE - p.shape[1]), -999.0, jnp.float32)
    cat = jnp.concatenate([p, neg], axis=1)
    m = jnp.max(cat, axis=1, keepdims=True)
    lse = m + jnp.log(jnp.sum(jnp.exp(cat - m), axis=1, keepdims=True))
    nx_ref[...] = cat - lse


def _run_stage3b(pir, interpret=False):
    G, PG = pir.shape
    return pl.pallas_call(
        _nx_body,
        grid=(1,),
        in_specs=[pl.BlockSpec((G, PG), lambda i: (0, 0))],
        out_specs=pl.BlockSpec((G, ACTION_SIZE), lambda i: (0, 0)),
        out_shape=jax.ShapeDtypeStruct((G, ACTION_SIZE), jnp.float32),
        interpret=interpret,
    )(pir)


def kernel(x, edge_index, edge_attr, batch, embed_table,
           w_ih_f, w_hh_f, b_ih_f, b_hh_f, w_ih_b, w_hh_b, b_ih_b, b_hh_b,
           pw1, pb1, pw2, pb2, pw3, pb3, pw4, pb4, vw1, vb1, vw2, vb2):
    f32 = jnp.float32
    N, XD = x.shape
    E = edge_index.shape[1]
    G = 200
    PG = N // G

    EMBD = embed_table.shape[1]
    embp = jnp.zeros((VOCAB_PAD, EMBD), f32).at[:embed_table.shape[0]].set(embed_table)
    W = jnp.zeros((128, 256), f32)
    W = W.at[0:32, 0:128].set(embp @ w_ih_f.T)
    W = W.at[32:64, 0:128].set(w_hh_f.T)
    W = W.at[64:96, 128:256].set(embp @ w_ih_b.T)
    W = W.at[96:128, 128:256].set(w_hh_b.T)
    bias = jnp.concatenate([b_ih_f + b_hh_f, b_ih_b + b_hh_b]).reshape(1, 256)

    data_out, data_in = _run_stage1(edge_attr, W, bias)

    src = edge_index[0]
    tgt = edge_index[1]
    zeros_hbm = jnp.zeros((N // 16, DROW), f32)
    sums_out, sums_in = _run_stage2(data_out, data_in, src, tgt, zeros_hbm, N)

    D2 = SND + 2 * H
    p1t = pw1.T
    P1x = p1t[0:XD]
    P1i = jnp.zeros((DROW, p1t.shape[1]), f32).at[0:D2].set(p1t[XD:XD + D2])
    P1o = jnp.zeros((DROW, p1t.shape[1]), f32).at[0:D2].set(p1t[XD + D2:])
    v1t = vw1.T
    V1x = v1t[0:XD]
    V1i = jnp.zeros((DROW, v1t.shape[1]), f32).at[0:D2].set(v1t[XD:XD + D2])
    V1o = jnp.zeros((DROW, v1t.shape[1]), f32).at[0:D2].set(v1t[XD + D2:])
    wts = [
        P1x, P1i, P1o, pb1.reshape(1, -1),
        pw2.T, pb2.reshape(1, -1), pw3.T, pb3.reshape(1, -1),
        pw4.T, pb4.reshape(1, -1),
        V1x, V1i, V1o, vb1.reshape(1, -1), vw2.T, vb2.reshape(1, -1),
    ]
    v, pi = _run_stage3(x, sums_in, sums_out, wts, G)

    new_x = _run_stage3b(pi.reshape(G, PG))
    return new_x, v


# speedup vs baseline: 28.6480x; 5.6756x over previous
import functools

import jax
import jax.numpy as jnp
from jax import lax
from jax.experimental import pallas as pl
from jax.experimental.pallas import tpu as pltpu
from jax.experimental.pallas import tpu_sc as plsc

MAX_LEN = 20
SND = 53
VOCAB_PAD = 32
H = 32
DROW = 128
CNT_COL = SND + 2 * H
ACTION_SIZE = 53

EBLK = 1280
GBLK = 40


def _lstm_body(eat_ref, w_ref, b_ref, out_ref, in_ref):
    f32 = jnp.float32
    toks = eat_ref[...]
    B = toks.shape[1]
    W = w_ref[...]
    bias = b_ref[...]
    iota = lax.broadcasted_iota(jnp.int32, (VOCAB_PAD, B), 0)
    h_f = jnp.zeros((H, B), f32)
    c_f = jnp.zeros((H, B), f32)
    h_b = jnp.zeros((H, B), f32)
    c_b = jnp.zeros((H, B), f32)
    acc_f = jnp.zeros((H, B), f32)
    acc_b = jnp.zeros((H, B), f32)
    for t in range(MAX_LEN):
        onef = (toks[t:t + 1, :] == iota).astype(f32)
        oneb = (toks[MAX_LEN - 1 - t:MAX_LEN - t, :] == iota).astype(f32)
        inp = jnp.concatenate([onef, h_f, oneb, h_b], axis=0)
        g = jnp.dot(W, inp, preferred_element_type=f32) + bias
        i_f = jnp.tanh(g[0:32]) * 0.5 + 0.5
        f_f = jnp.tanh(g[32:64]) * 0.5 + 0.5
        g_f = jnp.tanh(g[64:96])
        o_f = jnp.tanh(g[96:128]) * 0.5 + 0.5
        c_f = f_f * c_f + i_f * g_f
        h_f = o_f * jnp.tanh(c_f)
        i_b = jnp.tanh(g[128:160]) * 0.5 + 0.5
        f_b = jnp.tanh(g[160:192]) * 0.5 + 0.5
        g_b = jnp.tanh(g[192:224])
        o_b = jnp.tanh(g[224:256]) * 0.5 + 0.5
        c_b = f_b * c_b + i_b * g_b
        h_b = o_b * jnp.tanh(c_b)
        acc_f = acc_f + h_f
        acc_b = acc_b + h_b
    scale = 1.0 / MAX_LEN
    src_sn = toks[MAX_LEN:MAX_LEN + SND, :].astype(f32)
    tgt_sn = toks[MAX_LEN + SND:MAX_LEN + 2 * SND, :].astype(f32)
    ones = jnp.ones((1, B), f32)
    pad = jnp.zeros((DROW - CNT_COL - 1, B), f32)
    regex = jnp.concatenate([acc_f * scale, acc_b * scale], axis=0)
    out_t = jnp.concatenate([tgt_sn, regex, ones, pad], axis=0)
    in_t = jnp.concatenate([src_sn, regex, ones, pad], axis=0)
    out_ref[...] = jnp.swapaxes(out_t, 0, 1)
    in_ref[...] = jnp.swapaxes(in_t, 0, 1)


def _run_stage1(edge_attr_t, W, bias, interpret=False):
    A, E = edge_attr_t.shape
    grid = (E // EBLK,)
    return pl.pallas_call(
        _lstm_body,
        grid=grid,
        in_specs=[
            pl.BlockSpec((A, EBLK), lambda i: (0, i)),
            pl.BlockSpec((256, 128), lambda i: (0, 0)),
            pl.BlockSpec((256, 1), lambda i: (0, 0)),
        ],
        out_specs=[
            pl.BlockSpec((EBLK, DROW), lambda i: (i, 0)),
            pl.BlockSpec((EBLK, DROW), lambda i: (i, 0)),
        ],
        out_shape=[
            jax.ShapeDtypeStruct((E, DROW), jnp.float32),
            jax.ShapeDtypeStruct((E, DROW), jnp.float32),
        ],
        interpret=interpret,
    )(edge_attr_t, W, bias)


def _run_stage2(data_out, data_in, src, tgt, zeros_hbm, n_nodes):
    E = src.shape[0]
    NS = 16
    ET = E // NS
    CH = 128
    NFULL = ET // CH
    TAIL = ET - NFULL * CH
    TR = (n_nodes // NS) // 8 * 8
    TREM = n_nodes - TR * NS

    mesh = plsc.VectorSubcoreMesh(core_axis_name="c", subcore_axis_name="s")

    scratch = [
        pltpu.VMEM((CH, DROW), jnp.float32),
        pltpu.VMEM((CH,), jnp.int32),
        pltpu.VMEM_SHARED((n_nodes, DROW), jnp.float32),
    ]
    if TAIL:
        scratch += [
            pltpu.VMEM((TAIL, DROW), jnp.float32),
            pltpu.VMEM((TAIL,), jnp.int32),
        ]

    @functools.partial(
        pl.kernel,
        mesh=mesh,
        out_type=[
            jax.ShapeDtypeStruct((n_nodes, DROW), jnp.float32),
            jax.ShapeDtypeStruct((n_nodes, DROW), jnp.float32),
        ],
        scratch_types=scratch,
    )
    def sck(do_hbm, di_hbm, src_hbm, tgt_hbm, z_hbm, oo_hbm, oi_hbm,
            buf, idxv, acc, *tailbufs):
        c = lax.axis_index("c")
        s = lax.axis_index("s")
        pltpu.sync_copy(z_hbm.at[pl.ds(0, TR)], acc.at[pl.ds(s * TR, TR)])
        if TREM:
            @pl.when(s == NS - 1)
            def _():
                pltpu.sync_copy(z_hbm.at[pl.ds(0, TREM)],
                                acc.at[pl.ds(NS * TR, TREM)])
        plsc.subcore_barrier()

        def do_scatter(d_hbm, i_hbm):
            base = s * ET

            def body(k, carry):
                off = pl.multiple_of(base + k * CH, 8)
                pltpu.sync_copy(i_hbm.at[pl.ds(off, CH)], idxv)
                pltpu.sync_copy(d_hbm.at[pl.ds(off, CH)], buf)
                pltpu.sync_copy(buf, acc.at[idxv], add=True)
                return carry

            lax.fori_loop(0, NFULL, body, 0)
            if TAIL:
                tbuf, tidx = tailbufs
                off = base + NFULL * CH
                pltpu.sync_copy(i_hbm.at[pl.ds(off, TAIL)], tidx)
                pltpu.sync_copy(d_hbm.at[pl.ds(off, TAIL)], tbuf)
                pltpu.sync_copy(tbuf, acc.at[tidx], add=True)

        @pl.when(c == 0)
        def _():
            do_scatter(do_hbm, src_hbm)

        @pl.when(c == 1)
        def _():
            do_scatter(di_hbm, tgt_hbm)

        plsc.subcore_barrier()

        def writeout(o_hbm):
            pltpu.sync_copy(acc.at[pl.ds(s * TR, TR)], o_hbm.at[pl.ds(s * TR, TR)])
            if TREM:
                @pl.when(s == NS - 1)
                def _():
                    pltpu.sync_copy(acc.at[pl.ds(NS * TR, TREM)],
                                    o_hbm.at[pl.ds(NS * TR, TREM)])

        @pl.when(c == 0)
        def _():
            writeout(oo_hbm)

        @pl.when(c == 1)
        def _():
            writeout(oi_hbm)

    return sck(data_out, data_in, src, tgt, zeros_hbm)


def _mlp_body(x_ref, si_ref, so_ref,
              p1x_ref, p1i_ref, p1o_ref, pb1_ref,
              p2_ref, pb2_ref, p3_ref, pb3_ref, p4_ref, pb4_ref,
              v1x_ref, v1i_ref, v1o_ref, vb1_ref, v2_ref, vb2_ref,
              v_ref, pi_ref):
    f32 = jnp.float32
    x = x_ref[...]
    si = si_ref[...]
    so = so_ref[...]
    R = x.shape[0]
    cnt_i = jnp.maximum(si[:, CNT_COL:CNT_COL + 1], 1.0)
    cnt_o = jnp.maximum(so[:, CNT_COL:CNT_COL + 1], 1.0)
    inf = si / cnt_i
    outf = so / cnt_o

    def dot(a, b):
        return jnp.dot(a, b, preferred_element_type=f32)

    pi1 = jnp.maximum(dot(x, p1x_ref[...]) + dot(inf, p1i_ref[...])
                      + dot(outf, p1o_ref[...]) + pb1_ref[...], 0.0)
    pi2 = jnp.maximum(dot(pi1, p2_ref[...]) + pb2_ref[...], 0.0)
    pi3 = jnp.maximum(dot(pi2, p3_ref[...]) + pb3_ref[...], 0.0)
    pi4 = dot(pi3, p4_ref[...]) + pb4_ref[...]
    pi_ref[...] = pi4

    rows_per_graph = R // GBLK
    rg = lax.broadcasted_iota(jnp.int32, (GBLK, R), 1) // rows_per_graph
    gi = lax.broadcasted_iota(jnp.int32, (GBLK, R), 0)
    pool = (rg == gi).astype(f32) * (1.0 / rows_per_graph)
    mx = dot(pool, x)
    mi = dot(pool, inf)
    mo = dot(pool, outf)
    sv = jnp.maximum(dot(mx, v1x_ref[...]) + dot(mi, v1i_ref[...])
                     + dot(mo, v1o_ref[...]) + vb1_ref[...], 0.0)
    v_ref[...] = dot(sv, v2_ref[...]) + vb2_ref[...]


def _run_stage3(x, sums_in, sums_out, wts, n_graphs, interpret=False):
    N, XD = x.shape
    R = (N // n_graphs) * GBLK
    grid = (N // R,)

    def const_spec(a):
        return pl.BlockSpec(a.shape, lambda i: tuple(0 for _ in a.shape))

    in_specs = [
        pl.BlockSpec((R, XD), lambda i: (i, 0)),
        pl.BlockSpec((R, DROW), lambda i: (i, 0)),
        pl.BlockSpec((R, DROW), lambda i: (i, 0)),
    ] + [const_spec(w) for w in wts]
    return pl.pallas_call(
        _mlp_body,
        grid=grid,
        in_specs=in_specs,
        out_specs=[
            pl.BlockSpec((GBLK, 1), lambda i: (i, 0)),
            pl.BlockSpec((R, 1), lambda i: (i, 0)),
        ],
        out_shape=[
            jax.ShapeDtypeStruct((n_graphs, 1), jnp.float32),
            jax.ShapeDtypeStruct((N, 1), jnp.float32),
        ],
        interpret=interpret,
    )(x, sums_in, sums_out, *wts)


def _nx_body(pir_ref, nx_ref):
    p = pir_ref[...]
    Gb = p.shape[0]
    neg = jnp.full((Gb, ACTION_SIZE - p.shape[1]), -999.0, jnp.float32)
    cat = jnp.concatenate([p, neg], axis=1)
    m = jnp.max(cat, axis=1, keepdims=True)
    lse = m + jnp.log(jnp.sum(jnp.exp(cat - m), axis=1, keepdims=True))
    nx_ref[...] = cat - lse


def _run_stage3b(pir, interpret=False):
    G, PG = pir.shape
    return pl.pallas_call(
        _nx_body,
        grid=(1,),
        in_specs=[pl.BlockSpec((G, PG), lambda i: (0, 0))],
        out_specs=pl.BlockSpec((G, ACTION_SIZE), lambda i: (0, 0)),
        out_shape=jax.ShapeDtypeStruct((G, ACTION_SIZE), jnp.float32),
        interpret=interpret,
    )(pir)


def _stage1_weights(embed_table, w_ih_f, w_hh_f, b_ih_f, b_hh_f,
                    w_ih_b, w_hh_b, b_ih_b, b_hh_b):
    f32 = jnp.float32
    EMBD = embed_table.shape[1]
    embp = jnp.zeros((VOCAB_PAD, EMBD), f32).at[:embed_table.shape[0]].set(embed_table)
    W = jnp.zeros((256, 128), f32)
    W = W.at[0:128, 0:32].set((embp @ w_ih_f.T).T)
    W = W.at[0:128, 32:64].set(w_hh_f)
    W = W.at[128:256, 64:96].set((embp @ w_ih_b.T).T)
    W = W.at[128:256, 96:128].set(w_hh_b)
    bias = jnp.concatenate([b_ih_f + b_hh_f, b_ih_b + b_hh_b]).reshape(256, 1)
    gate_row = jnp.arange(256) % 128
    is_sig = (gate_row < 64) | (gate_row >= 96)
    sc = jnp.where(is_sig, 0.5, 1.0).astype(f32)
    return W * sc[:, None], bias * sc[:, None]

def kernel(x, edge_index, edge_attr, batch, embed_table,
           w_ih_f, w_hh_f, b_ih_f, b_hh_f, w_ih_b, w_hh_b, b_ih_b, b_hh_b,
           pw1, pb1, pw2, pb2, pw3, pb3, pw4, pb4, vw1, vb1, vw2, vb2):
    f32 = jnp.float32
    N, XD = x.shape
    E = edge_index.shape[1]
    G = 200
    PG = N // G

    W, bias = _stage1_weights(embed_table, w_ih_f, w_hh_f, b_ih_f, b_hh_f,
                              w_ih_b, w_hh_b, b_ih_b, b_hh_b)
    data_out, data_in = _run_stage1(edge_attr.T, W, bias)

    src = edge_index[0]
    tgt = edge_index[1]
    zeros_hbm = jnp.zeros((N // 16, DROW), f32)
    sums_out, sums_in = _run_stage2(data_out, data_in, src, tgt, zeros_hbm, N)

    D2 = SND + 2 * H
    p1t = pw1.T
    P1x = p1t[0:XD]
    P1i = jnp.zeros((DROW, p1t.shape[1]), f32).at[0:D2].set(p1t[XD:XD + D2])
    P1o = jnp.zeros((DROW, p1t.shape[1]), f32).at[0:D2].set(p1t[XD + D2:])
    v1t = vw1.T
    V1x = v1t[0:XD]
    V1i = jnp.zeros((DROW, v1t.shape[1]), f32).at[0:D2].set(v1t[XD:XD + D2])
    V1o = jnp.zeros((DROW, v1t.shape[1]), f32).at[0:D2].set(v1t[XD + D2:])
    wts = [
        P1x, P1i, P1o, pb1.reshape(1, -1),
        pw2.T, pb2.reshape(1, -1), pw3.T, pb3.reshape(1, -1),
        pw4.T, pb4.reshape(1, -1),
        V1x, V1i, V1o, vb1.reshape(1, -1), vw2.T, vb2.reshape(1, -1),
    ]
    v, pi = _run_stage3(x, sums_in, sums_out, wts, G)

    new_x = _run_stage3b(pi.reshape(G, PG))
    return new_x, v


# speedup vs baseline: 30.4099x; 1.0615x over previous
import functools

import jax
import jax.numpy as jnp
from jax import lax
from jax.experimental import pallas as pl
from jax.experimental.pallas import tpu as pltpu
from jax.experimental.pallas import tpu_sc as plsc

MAX_LEN = 20
SND = 53
VOCAB_PAD = 32
H = 32
DROW = 128
CNT_COL = SND + 2 * H
ACTION_SIZE = 53

EBLK = 1280
GBLK = 40


def _lstm_body(ea_ref, w_ref, out_ref, in_ref):
    f32 = jnp.float32
    ea = ea_ref[...]
    B = ea.shape[0]
    toks = jnp.swapaxes(ea[:, :MAX_LEN], 0, 1)
    W = w_ref[...]
    iota = lax.broadcasted_iota(jnp.int32, (VOCAB_PAD, B), 0)
    m31 = iota == (VOCAB_PAD - 1)
    h_f = jnp.zeros((H, B), f32)
    c_f = jnp.zeros((H, B), f32)
    h_b = jnp.zeros((H, B), f32)
    c_b = jnp.zeros((H, B), f32)
    acc_f = jnp.zeros((H, B), f32)
    acc_b = jnp.zeros((H, B), f32)
    for t in range(MAX_LEN):
        onef = ((toks[t:t + 1, :] == iota) | m31).astype(f32)
        oneb = (toks[MAX_LEN - 1 - t:MAX_LEN - t, :] == iota).astype(f32)
        inp = jnp.concatenate([onef, h_f, oneb, h_b], axis=0)
        g = jnp.dot(W, inp, preferred_element_type=f32)
        i_f = jnp.tanh(g[0:32]) * 0.5 + 0.5
        f_f = jnp.tanh(g[32:64]) * 0.5 + 0.5
        g_f = jnp.tanh(g[64:96])
        o_f = jnp.tanh(g[96:128]) * 0.5 + 0.5
        c_f = f_f * c_f + i_f * g_f
        h_f = o_f * jnp.tanh(c_f)
        i_b = jnp.tanh(g[128:160]) * 0.5 + 0.5
        f_b = jnp.tanh(g[160:192]) * 0.5 + 0.5
        g_b = jnp.tanh(g[192:224])
        o_b = jnp.tanh(g[224:256]) * 0.5 + 0.5
        c_b = f_b * c_b + i_b * g_b
        h_b = o_b * jnp.tanh(c_b)
        acc_f = acc_f + h_f
        acc_b = acc_b + h_b
    scale = 1.0 / MAX_LEN
    regex_t = jnp.concatenate([acc_f * scale, acc_b * scale], axis=0)
    regex = jnp.swapaxes(regex_t, 0, 1)
    src_sn = ea[:, MAX_LEN:MAX_LEN + SND].astype(f32)
    tgt_sn = ea[:, MAX_LEN + SND:MAX_LEN + 2 * SND].astype(f32)
    ones = jnp.ones((B, 1), f32)
    pad = jnp.zeros((B, DROW - CNT_COL - 1), f32)
    out_ref[...] = jnp.concatenate([tgt_sn, regex, ones, pad], axis=1)
    in_ref[...] = jnp.concatenate([src_sn, regex, ones, pad], axis=1)


def _run_stage1(edge_attr, W, interpret=False):
    E, A = edge_attr.shape
    grid = (E // EBLK,)
    return pl.pallas_call(
        _lstm_body,
        grid=grid,
        in_specs=[
            pl.BlockSpec((EBLK, A), lambda i: (i, 0)),
            pl.BlockSpec((256, 128), lambda i: (0, 0)),
        ],
        out_specs=[
            pl.BlockSpec((EBLK, DROW), lambda i: (i, 0)),
            pl.BlockSpec((EBLK, DROW), lambda i: (i, 0)),
        ],
        out_shape=[
            jax.ShapeDtypeStruct((E, DROW), jnp.float32),
            jax.ShapeDtypeStruct((E, DROW), jnp.float32),
        ],
        interpret=interpret,
    )(edge_attr, W)


def _run_stage2(data_out, data_in, src, tgt, zeros_hbm, n_nodes):
    E = src.shape[0]
    NS = 16
    ET = E // NS
    CH = 128
    NFULL = ET // CH
    TAIL = ET - NFULL * CH
    TR = (n_nodes // NS) // 8 * 8
    TREM = n_nodes - TR * NS

    mesh = plsc.VectorSubcoreMesh(core_axis_name="c", subcore_axis_name="s")

    scratch = [
        pltpu.VMEM((CH, DROW), jnp.float32),
        pltpu.VMEM((CH,), jnp.int32),
        pltpu.VMEM_SHARED((n_nodes, DROW), jnp.float32),
    ]
    if TAIL:
        scratch += [
            pltpu.VMEM((TAIL, DROW), jnp.float32),
            pltpu.VMEM((TAIL,), jnp.int32),
        ]

    @functools.partial(
        pl.kernel,
        mesh=mesh,
        out_type=[
            jax.ShapeDtypeStruct((n_nodes, DROW), jnp.float32),
            jax.ShapeDtypeStruct((n_nodes, DROW), jnp.float32),
        ],
        scratch_types=scratch,
    )
    def sck(do_hbm, di_hbm, src_hbm, tgt_hbm, z_hbm, oo_hbm, oi_hbm,
            buf, idxv, acc, *tailbufs):
        c = lax.axis_index("c")
        s = lax.axis_index("s")
        pltpu.sync_copy(z_hbm.at[pl.ds(0, TR)], acc.at[pl.ds(s * TR, TR)])
        if TREM:
            @pl.when(s == NS - 1)
            def _():
                pltpu.sync_copy(z_hbm.at[pl.ds(0, TREM)],
                                acc.at[pl.ds(NS * TR, TREM)])
        plsc.subcore_barrier()

        def do_scatter(d_hbm, i_hbm):
            base = s * ET

            def body(k, carry):
                off = pl.multiple_of(base + k * CH, 8)
                pltpu.sync_copy(i_hbm.at[pl.ds(off, CH)], idxv)
                pltpu.sync_copy(d_hbm.at[pl.ds(off, CH)], buf)
                pltpu.sync_copy(buf, acc.at[idxv], add=True)
                return carry

            lax.fori_loop(0, NFULL, body, 0)
            if TAIL:
                tbuf, tidx = tailbufs
                off = base + NFULL * CH
                pltpu.sync_copy(i_hbm.at[pl.ds(off, TAIL)], tidx)
                pltpu.sync_copy(d_hbm.at[pl.ds(off, TAIL)], tbuf)
                pltpu.sync_copy(tbuf, acc.at[tidx], add=True)

        @pl.when(c == 0)
        def _():
            do_scatter(do_hbm, src_hbm)

        @pl.when(c == 1)
        def _():
            do_scatter(di_hbm, tgt_hbm)

        plsc.subcore_barrier()

        def writeout(o_hbm):
            pltpu.sync_copy(acc.at[pl.ds(s * TR, TR)], o_hbm.at[pl.ds(s * TR, TR)])
            if TREM:
                @pl.when(s == NS - 1)
                def _():
                    pltpu.sync_copy(acc.at[pl.ds(NS * TR, TREM)],
                                    o_hbm.at[pl.ds(NS * TR, TREM)])

        @pl.when(c == 0)
        def _():
            writeout(oo_hbm)

        @pl.when(c == 1)
        def _():
            writeout(oi_hbm)

    return sck(data_out, data_in, src, tgt, zeros_hbm)


def _mlp_body(x_ref, si_ref, so_ref,
              p1x_ref, p1i_ref, p1o_ref, pb1_ref,
              p2_ref, pb2_ref, p3_ref, pb3_ref, p4_ref, pb4_ref,
              v1x_ref, v1i_ref, v1o_ref, vb1_ref, v2_ref, vb2_ref,
              v_ref, pi_ref):
    f32 = jnp.float32
    x = x_ref[...]
    si = si_ref[...]
    so = so_ref[...]
    R = x.shape[0]
    cnt_i = jnp.maximum(si[:, CNT_COL:CNT_COL + 1], 1.0)
    cnt_o = jnp.maximum(so[:, CNT_COL:CNT_COL + 1], 1.0)
    inf = si / cnt_i
    outf = so / cnt_o

    def dot(a, b):
        return jnp.dot(a, b, preferred_element_type=f32)

    pi1 = jnp.maximum(dot(x, p1x_ref[...]) + dot(inf, p1i_ref[...])
                      + dot(outf, p1o_ref[...]) + pb1_ref[...], 0.0)
    pi2 = jnp.maximum(dot(pi1, p2_ref[...]) + pb2_ref[...], 0.0)
    pi3 = jnp.maximum(dot(pi2, p3_ref[...]) + pb3_ref[...], 0.0)
    pi4 = dot(pi3, p4_ref[...]) + pb4_ref[...]
    pi_ref[...] = pi4

    rows_per_graph = R // GBLK
    rg = lax.broadcasted_iota(jnp.int32, (GBLK, R), 1) // rows_per_graph
    gi = lax.broadcasted_iota(jnp.int32, (GBLK, R), 0)
    pool = (rg == gi).astype(f32) * (1.0 / rows_per_graph)
    mx = dot(pool, x)
    mi = dot(pool, inf)
    mo = dot(pool, outf)
    sv = jnp.maximum(dot(mx, v1x_ref[...]) + dot(mi, v1i_ref[...])
                     + dot(mo, v1o_ref[...]) + vb1_ref[...], 0.0)
    v_ref[...] = dot(sv, v2_ref[...]) + vb2_ref[...]


def _run_stage3(x, sums_in, sums_out, wts, n_graphs, interpret=False):
    N, XD = x.shape
    R = (N // n_graphs) * GBLK
    grid = (N // R,)

    def const_spec(a):
        return pl.BlockSpec(a.shape, lambda i: tuple(0 for _ in a.shape))

    in_specs = [
        pl.BlockSpec((R, XD), lambda i: (i, 0)),
        pl.BlockSpec((R, DROW), lambda i: (i, 0)),
        pl.BlockSpec((R, DROW), lambda i: (i, 0)),
    ] + [const_spec(w) for w in wts]
    return pl.pallas_call(
        _mlp_body,
        grid=grid,
        in_specs=in_specs,
        out_specs=[
            pl.BlockSpec((GBLK, 1), lambda i: (i, 0)),
            pl.BlockSpec((R, 1), lambda i: (i, 0)),
        ],
        out_shape=[
            jax.ShapeDtypeStruct((n_graphs, 1), jnp.float32),
            jax.ShapeDtypeStruct((N, 1), jnp.float32),
        ],
        interpret=interpret,
    )(x, sums_in, sums_out, *wts)


def _nx_body(pir_ref, nx_ref):
    p = pir_ref[...]
    Gb = p.shape[0]
    neg = jnp.full((Gb, ACTION_SIZE - p.shape[1]), -999.0, jnp.float32)
    cat = jnp.concatenate([p, neg], axis=1)
    m = jnp.max(cat, axis=1, keepdims=True)
    lse = m + jnp.log(jnp.sum(jnp.exp(cat - m), axis=1, keepdims=True))
    nx_ref[...] = cat - lse


def _run_stage3b(pir, interpret=False):
    G, PG = pir.shape
    return pl.pallas_call(
        _nx_body,
        grid=(1,),
        in_specs=[pl.BlockSpec((G, PG), lambda i: (0, 0))],
        out_specs=pl.BlockSpec((G, ACTION_SIZE), lambda i: (0, 0)),
        out_shape=jax.ShapeDtypeStruct((G, ACTION_SIZE), jnp.float32),
        interpret=interpret,
    )(pir)


def _stage1_weights(embed_table, w_ih_f, w_hh_f, b_ih_f, b_hh_f,
                    w_ih_b, w_hh_b, b_ih_b, b_hh_b):
    f32 = jnp.float32
    EMBD = embed_table.shape[1]
    embp = jnp.zeros((VOCAB_PAD, EMBD), f32).at[:embed_table.shape[0]].set(embed_table)
    W = jnp.zeros((256, 128), f32)
    W = W.at[0:128, 0:32].set((embp @ w_ih_f.T).T)
    W = W.at[0:128, 32:64].set(w_hh_f)
    W = W.at[128:256, 64:96].set((embp @ w_ih_b.T).T)
    W = W.at[128:256, 96:128].set(w_hh_b)
    bias = jnp.concatenate([b_ih_f + b_hh_f, b_ih_b + b_hh_b])
    W = W.at[:, VOCAB_PAD - 1].set(bias)
    gate_row = jnp.arange(256) % 128
    is_sig = (gate_row < 64) | (gate_row >= 96)
    sc = jnp.where(is_sig, 0.5, 1.0).astype(f32)
    return W * sc[:, None]

def kernel(x, edge_index, edge_attr, batch, embed_table,
           w_ih_f, w_hh_f, b_ih_f, b_hh_f, w_ih_b, w_hh_b, b_ih_b, b_hh_b,
           pw1, pb1, pw2, pb2, pw3, pb3, pw4, pb4, vw1, vb1, vw2, vb2):
    f32 = jnp.float32
    N, XD = x.shape
    E = edge_index.shape[1]
    G = 200
    PG = N // G

    W = _stage1_weights(embed_table, w_ih_f, w_hh_f, b_ih_f, b_hh_f,
                        w_ih_b, w_hh_b, b_ih_b, b_hh_b)
    data_out, data_in = _run_stage1(edge_attr, W)

    src = edge_index[0]
    tgt = edge_index[1]
    zeros_hbm = jnp.zeros((N // 16, DROW), f32)
    sums_out, sums_in = _run_stage2(data_out, data_in, src, tgt, zeros_hbm, N)

    D2 = SND + 2 * H
    p1t = pw1.T
    P1x = p1t[0:XD]
    P1i = jnp.zeros((DROW, p1t.shape[1]), f32).at[0:D2].set(p1t[XD:XD + D2])
    P1o = jnp.zeros((DROW, p1t.shape[1]), f32).at[0:D2].set(p1t[XD + D2:])
    v1t = vw1.T
    V1x = v1t[0:XD]
    V1i = jnp.zeros((DROW, v1t.shape[1]), f32).at[0:D2].set(v1t[XD:XD + D2])
    V1o = jnp.zeros((DROW, v1t.shape[1]), f32).at[0:D2].set(v1t[XD + D2:])
    wts = [
        P1x, P1i, P1o, pb1.reshape(1, -1),
        pw2.T, pb2.reshape(1, -1), pw3.T, pb3.reshape(1, -1),
        pw4.T, pb4.reshape(1, -1),
        V1x, V1i, V1o, vb1.reshape(1, -1), vw2.T, vb2.reshape(1, -1),
    ]
    v, pi = _run_stage3(x, sums_in, sums_out, wts, G)

    new_x = _run_stage3b(pi.reshape(G, PG))
    return new_x, v


# speedup vs baseline: 31.4443x; 1.0340x over previous
import functools

import jax
import jax.numpy as jnp
from jax import lax
from jax.experimental import pallas as pl
from jax.experimental.pallas import tpu as pltpu
from jax.experimental.pallas import tpu_sc as plsc

MAX_LEN = 20
SND = 53
VOCAB_PAD = 32
H = 32
DROW = 128
CNT_COL = SND + 2 * H
ACTION_SIZE = 53

EBLK = 1280
NHALF = 2
GBLK = 40


def _lstm_body(ea_ref, w_ref, out_ref, in_ref):
    f32 = jnp.float32
    ea = ea_ref[...]
    B = ea.shape[0]
    toks = jnp.swapaxes(ea[:, :MAX_LEN], 0, 1)
    W = w_ref[...]
    iota = lax.broadcasted_iota(jnp.int32, (VOCAB_PAD, B), 0)
    m31 = iota == (VOCAB_PAD - 1)
    h_f = jnp.zeros((H, B), f32)
    c_f = jnp.zeros((H, B), f32)
    h_b = jnp.zeros((H, B), f32)
    c_b = jnp.zeros((H, B), f32)
    acc_f = jnp.zeros((H, B), f32)
    acc_b = jnp.zeros((H, B), f32)
    for t in range(MAX_LEN):
        onef = ((toks[t:t + 1, :] == iota) | m31).astype(f32)
        oneb = (toks[MAX_LEN - 1 - t:MAX_LEN - t, :] == iota).astype(f32)
        inp = jnp.concatenate([onef, h_f, oneb, h_b], axis=0)
        g = jnp.dot(W, inp, preferred_element_type=f32)
        i_f = jnp.tanh(g[0:32]) * 0.5 + 0.5
        f_f = jnp.tanh(g[32:64]) * 0.5 + 0.5
        g_f = jnp.tanh(g[64:96])
        o_f = jnp.tanh(g[96:128]) * 0.5 + 0.5
        c_f = f_f * c_f + i_f * g_f
        h_f = o_f * jnp.tanh(c_f)
        i_b = jnp.tanh(g[128:160]) * 0.5 + 0.5
        f_b = jnp.tanh(g[160:192]) * 0.5 + 0.5
        g_b = jnp.tanh(g[192:224])
        o_b = jnp.tanh(g[224:256]) * 0.5 + 0.5
        c_b = f_b * c_b + i_b * g_b
        h_b = o_b * jnp.tanh(c_b)
        acc_f = acc_f + h_f
        acc_b = acc_b + h_b
    scale = 1.0 / MAX_LEN
    regex_t = jnp.concatenate([acc_f * scale, acc_b * scale], axis=0)
    regex = jnp.swapaxes(regex_t, 0, 1)
    src_sn = ea[:, MAX_LEN:MAX_LEN + SND].astype(f32)
    tgt_sn = ea[:, MAX_LEN + SND:MAX_LEN + 2 * SND].astype(f32)
    ones = jnp.ones((B, 1), f32)
    pad = jnp.zeros((B, DROW - CNT_COL - 1), f32)
    out_ref[...] = jnp.concatenate([tgt_sn, regex, ones, pad], axis=1)
    in_ref[...] = jnp.concatenate([src_sn, regex, ones, pad], axis=1)


def _run_stage1(edge_attr, W, interpret=False):
    E, A = edge_attr.shape
    grid = (E // EBLK,)
    return pl.pallas_call(
        _lstm_body,
        grid=grid,
        in_specs=[
            pl.BlockSpec((EBLK, A), lambda i: (i, 0)),
            pl.BlockSpec((256, 128), lambda i: (0, 0)),
        ],
        out_specs=[
            pl.BlockSpec((EBLK, DROW), lambda i: (i, 0)),
            pl.BlockSpec((EBLK, DROW), lambda i: (i, 0)),
        ],
        out_shape=[
            jax.ShapeDtypeStruct((E, DROW), jnp.float32),
            jax.ShapeDtypeStruct((E, DROW), jnp.float32),
        ],
        interpret=interpret,
    )(edge_attr, W)


def _run_stage2(data_out, data_in, src, tgt, zeros_hbm, n_nodes):
    E = src.shape[0]
    NS = 16
    ET = E // NS
    CH = 128
    NFULL = ET // CH
    TAIL = ET - NFULL * CH
    TR = (n_nodes // NS) // 8 * 8
    TREM = n_nodes - TR * NS

    mesh = plsc.VectorSubcoreMesh(core_axis_name="c", subcore_axis_name="s")

    scratch = [
        pltpu.VMEM((CH, DROW), jnp.float32),
        pltpu.VMEM((CH,), jnp.int32),
        pltpu.VMEM_SHARED((n_nodes, DROW), jnp.float32),
    ]
    if TAIL:
        scratch += [
            pltpu.VMEM((TAIL, DROW), jnp.float32),
            pltpu.VMEM((TAIL,), jnp.int32),
        ]

    @functools.partial(
        pl.kernel,
        mesh=mesh,
        out_type=[
            jax.ShapeDtypeStruct((n_nodes, DROW), jnp.float32),
            jax.ShapeDtypeStruct((n_nodes, DROW), jnp.float32),
        ],
        scratch_types=scratch,
    )
    def sck(do_hbm, di_hbm, src_hbm, tgt_hbm, z_hbm, oo_hbm, oi_hbm,
            buf, idxv, acc, *tailbufs):
        c = lax.axis_index("c")
        s = lax.axis_index("s")
        pltpu.sync_copy(z_hbm.at[pl.ds(0, TR)], acc.at[pl.ds(s * TR, TR)])
        if TREM:
            @pl.when(s == NS - 1)
            def _():
                pltpu.sync_copy(z_hbm.at[pl.ds(0, TREM)],
                                acc.at[pl.ds(NS * TR, TREM)])
        plsc.subcore_barrier()

        def do_scatter(d_hbm, i_hbm):
            base = s * ET

            def body(k, carry):
                off = pl.multiple_of(base + k * CH, 8)
                pltpu.sync_copy(i_hbm.at[pl.ds(off, CH)], idxv)
                pltpu.sync_copy(d_hbm.at[pl.ds(off, CH)], buf)
                pltpu.sync_copy(buf, acc.at[idxv], add=True)
                return carry

            lax.fori_loop(0, NFULL, body, 0)
            if TAIL:
                tbuf, tidx = tailbufs
                off = base + NFULL * CH
                pltpu.sync_copy(i_hbm.at[pl.ds(off, TAIL)], tidx)
                pltpu.sync_copy(d_hbm.at[pl.ds(off, TAIL)], tbuf)
                pltpu.sync_copy(tbuf, acc.at[tidx], add=True)

        @pl.when(c == 0)
        def _():
            do_scatter(do_hbm, src_hbm)

        @pl.when(c == 1)
        def _():
            do_scatter(di_hbm, tgt_hbm)

        plsc.subcore_barrier()

        def writeout(o_hbm):
            pltpu.sync_copy(acc.at[pl.ds(s * TR, TR)], o_hbm.at[pl.ds(s * TR, TR)])
            if TREM:
                @pl.when(s == NS - 1)
                def _():
                    pltpu.sync_copy(acc.at[pl.ds(NS * TR, TREM)],
                                    o_hbm.at[pl.ds(NS * TR, TREM)])

        @pl.when(c == 0)
        def _():
            writeout(oo_hbm)

        @pl.when(c == 1)
        def _():
            writeout(oi_hbm)

    return sck(data_out, data_in, src, tgt, zeros_hbm)


def _mlp_body(x_ref, *refs):
    si_refs = refs[0:NHALF]
    so_refs = refs[NHALF:2 * NHALF]
    (p1x_ref, p1i_ref, p1o_ref, pb1_ref,
     p2_ref, pb2_ref, p3_ref, pb3_ref, p4_ref, pb4_ref,
     v1x_ref, v1i_ref, v1o_ref, vb1_ref, v2_ref, vb2_ref,
     v_ref, pi_ref) = refs[2 * NHALF:]
    f32 = jnp.float32
    x = x_ref[...]
    si = si_refs[0][...]
    so = so_refs[0][...]
    for r in si_refs[1:]:
        si = si + r[...]
    for r in so_refs[1:]:
        so = so + r[...]
    R = x.shape[0]
    cnt_i = jnp.maximum(si[:, CNT_COL:CNT_COL + 1], 1.0)
    cnt_o = jnp.maximum(so[:, CNT_COL:CNT_COL + 1], 1.0)
    inf = si / cnt_i
    outf = so / cnt_o

    def dot(a, b):
        return jnp.dot(a, b, preferred_element_type=f32)

    pi1 = jnp.maximum(dot(x, p1x_ref[...]) + dot(inf, p1i_ref[...])
                      + dot(outf, p1o_ref[...]) + pb1_ref[...], 0.0)
    pi2 = jnp.maximum(dot(pi1, p2_ref[...]) + pb2_ref[...], 0.0)
    pi3 = jnp.maximum(dot(pi2, p3_ref[...]) + pb3_ref[...], 0.0)
    pi4 = dot(pi3, p4_ref[...]) + pb4_ref[...]
    pi_ref[...] = pi4

    rows_per_graph = R // GBLK
    rg = lax.broadcasted_iota(jnp.int32, (GBLK, R), 1) // rows_per_graph
    gi = lax.broadcasted_iota(jnp.int32, (GBLK, R), 0)
    pool = (rg == gi).astype(f32) * (1.0 / rows_per_graph)
    mx = dot(pool, x)
    mi = dot(pool, inf)
    mo = dot(pool, outf)
    sv = jnp.maximum(dot(mx, v1x_ref[...]) + dot(mi, v1i_ref[...])
                     + dot(mo, v1o_ref[...]) + vb1_ref[...], 0.0)
    v_ref[...] = dot(sv, v2_ref[...]) + vb2_ref[...]


def _run_stage3(x, sums_in_list, sums_out_list, wts, n_graphs, interpret=False):
    N, XD = x.shape
    R = (N // n_graphs) * GBLK
    grid = (N // R,)

    def const_spec(a):
        return pl.BlockSpec(a.shape, lambda i: tuple(0 for _ in a.shape))

    in_specs = [
        pl.BlockSpec((R, XD), lambda i: (i, 0)),
    ] + [pl.BlockSpec((R, DROW), lambda i: (i, 0))] * (2 * NHALF) \
      + [const_spec(w) for w in wts]
    return pl.pallas_call(
        _mlp_body,
        grid=grid,
        in_specs=in_specs,
        out_specs=[
            pl.BlockSpec((GBLK, 1), lambda i: (i, 0)),
            pl.BlockSpec((R, 1), lambda i: (i, 0)),
        ],
        out_shape=[
            jax.ShapeDtypeStruct((n_graphs, 1), jnp.float32),
            jax.ShapeDtypeStruct((N, 1), jnp.float32),
        ],
        interpret=interpret,
    )(x, *sums_in_list, *sums_out_list, *wts)


def _nx_body(pir_ref, nx_ref):
    p = pir_ref[...]
    Gb = p.shape[0]
    neg = jnp.full((Gb, ACTION_SIZE - p.shape[1]), -999.0, jnp.float32)
    cat = jnp.concatenate([p, neg], axis=1)
    m = jnp.max(cat, axis=1, keepdims=True)
    lse = m + jnp.log(jnp.sum(jnp.exp(cat - m), axis=1, keepdims=True))
    nx_ref[...] = cat - lse


def _run_stage3b(pir, interpret=False):
    G, PG = pir.shape
    return pl.pallas_call(
        _nx_body,
        grid=(1,),
        in_specs=[pl.BlockSpec((G, PG), lambda i: (0, 0))],
        out_specs=pl.BlockSpec((G, ACTION_SIZE), lambda i: (0, 0)),
        out_shape=jax.ShapeDtypeStruct((G, ACTION_SIZE), jnp.float32),
        interpret=interpret,
    )(pir)


def _stage1_weights(embed_table, w_ih_f, w_hh_f, b_ih_f, b_hh_f,
                    w_ih_b, w_hh_b, b_ih_b, b_hh_b):
    f32 = jnp.float32
    EMBD = embed_table.shape[1]
    embp = jnp.zeros((VOCAB_PAD, EMBD), f32).at[:embed_table.shape[0]].set(embed_table)
    W = jnp.zeros((256, 128), f32)
    W = W.at[0:128, 0:32].set((embp @ w_ih_f.T).T)
    W = W.at[0:128, 32:64].set(w_hh_f)
    W = W.at[128:256, 64:96].set((embp @ w_ih_b.T).T)
    W = W.at[128:256, 96:128].set(w_hh_b)
    bias = jnp.concatenate([b_ih_f + b_hh_f, b_ih_b + b_hh_b])
    W = W.at[:, VOCAB_PAD - 1].set(bias)
    gate_row = jnp.arange(256) % 128
    is_sig = (gate_row < 64) | (gate_row >= 96)
    sc = jnp.where(is_sig, 0.5, 1.0).astype(f32)
    return W * sc[:, None]

def kernel(x, edge_index, edge_attr, batch, embed_table,
           w_ih_f, w_hh_f, b_ih_f, b_hh_f, w_ih_b, w_hh_b, b_ih_b, b_hh_b,
           pw1, pb1, pw2, pb2, pw3, pb3, pw4, pb4, vw1, vb1, vw2, vb2):
    f32 = jnp.float32
    N, XD = x.shape
    E = edge_index.shape[1]
    G = 200
    PG = N // G

    W = _stage1_weights(embed_table, w_ih_f, w_hh_f, b_ih_f, b_hh_f,
                        w_ih_b, w_hh_b, b_ih_b, b_hh_b)
    zeros_hbm = jnp.zeros((N // 16, DROW), f32)
    nb = E // EBLK
    bounds = [0]
    for h in range(NHALF):
        bounds.append(bounds[-1] + (nb // NHALF + (1 if h < nb % NHALF else 0)) * EBLK)
    sums_out_list, sums_in_list = [], []
    for h in range(NHALF):
        sl = slice(bounds[h], bounds[h + 1])
        data_out, data_in = _run_stage1(edge_attr[sl], W)
        so_h, si_h = _run_stage2(data_out, data_in, edge_index[0, sl],
                                 edge_index[1, sl], zeros_hbm, N)
        sums_out_list.append(so_h)
        sums_in_list.append(si_h)

    D2 = SND + 2 * H
    p1t = pw1.T
    P1x = p1t[0:XD]
    P1i = jnp.zeros((DROW, p1t.shape[1]), f32).at[0:D2].set(p1t[XD:XD + D2])
    P1o = jnp.zeros((DROW, p1t.shape[1]), f32).at[0:D2].set(p1t[XD + D2:])
    v1t = vw1.T
    V1x = v1t[0:XD]
    V1i = jnp.zeros((DROW, v1t.shape[1]), f32).at[0:D2].set(v1t[XD:XD + D2])
    V1o = jnp.zeros((DROW, v1t.shape[1]), f32).at[0:D2].set(v1t[XD + D2:])
    wts = [
        P1x, P1i, P1o, pb1.reshape(1, -1),
        pw2.T, pb2.reshape(1, -1), pw3.T, pb3.reshape(1, -1),
        pw4.T, pb4.reshape(1, -1),
        V1x, V1i, V1o, vb1.reshape(1, -1), vw2.T, vb2.reshape(1, -1),
    ]
    v, pi = _run_stage3(x, sums_in_list, sums_out_list, wts, G)

    new_x = _run_stage3b(pi.reshape(G, PG))
    return new_x, v


# speedup vs baseline: 34.5287x; 1.0981x over previous
import functools

import jax
import jax.numpy as jnp
from jax import lax
from jax.experimental import pallas as pl
from jax.experimental.pallas import tpu as pltpu
from jax.experimental.pallas import tpu_sc as plsc

MAX_LEN = 20
SND = 53
VOCAB_PAD = 32
H = 32
DROW = 128
CNT_COL = SND + 2 * H
ACTION_SIZE = 53

EBLK = 1280
NHALF = 3
GBLK = 40


def _lstm_body(ea_ref, w_ref, out_ref, in_ref):
    f32 = jnp.float32
    ea = ea_ref[...]
    B = ea.shape[0]
    toks = jnp.swapaxes(ea[:, :MAX_LEN], 0, 1)
    W = w_ref[...]
    iota = lax.broadcasted_iota(jnp.int32, (VOCAB_PAD, B), 0)
    m31 = iota == (VOCAB_PAD - 1)
    h_f = jnp.zeros((H, B), f32)
    c_f = jnp.zeros((H, B), f32)
    h_b = jnp.zeros((H, B), f32)
    c_b = jnp.zeros((H, B), f32)
    acc_f = jnp.zeros((H, B), f32)
    acc_b = jnp.zeros((H, B), f32)
    ohs = [((toks[t:t + 1, :] == iota) | m31).astype(f32) for t in range(MAX_LEN)]
    for t in range(MAX_LEN):
        onef = ohs[t]
        oneb = ohs[MAX_LEN - 1 - t]
        inp = jnp.concatenate([onef, h_f, oneb, h_b], axis=0)
        g = jnp.dot(W, inp, preferred_element_type=f32)
        i_f = jnp.tanh(g[0:32]) * 0.5 + 0.5
        f_f = jnp.tanh(g[32:64]) * 0.5 + 0.5
        g_f = jnp.tanh(g[64:96])
        o_f = jnp.tanh(g[96:128]) * 0.5 + 0.5
        c_f = f_f * c_f + i_f * g_f
        h_f = o_f * jnp.tanh(c_f)
        i_b = jnp.tanh(g[128:160]) * 0.5 + 0.5
        f_b = jnp.tanh(g[160:192]) * 0.5 + 0.5
        g_b = jnp.tanh(g[192:224])
        o_b = jnp.tanh(g[224:256]) * 0.5 + 0.5
        c_b = f_b * c_b + i_b * g_b
        h_b = o_b * jnp.tanh(c_b)
        acc_f = acc_f + h_f
        acc_b = acc_b + h_b
    scale = 1.0 / MAX_LEN
    regex_t = jnp.concatenate([acc_f * scale, acc_b * scale], axis=0)
    regex = jnp.swapaxes(regex_t, 0, 1)
    src_sn = ea[:, MAX_LEN:MAX_LEN + SND].astype(f32)
    tgt_sn = ea[:, MAX_LEN + SND:MAX_LEN + 2 * SND].astype(f32)
    ones = jnp.ones((B, 1), f32)
    pad = jnp.zeros((B, DROW - CNT_COL - 1), f32)
    out_ref[...] = jnp.concatenate([tgt_sn, regex, ones, pad], axis=1)
    in_ref[...] = jnp.concatenate([src_sn, regex, ones, pad], axis=1)


def _run_stage1(edge_attr, W, blk0=0, nblk=None, interpret=False):
    E, A = edge_attr.shape
    if nblk is None:
        nblk = E // EBLK - blk0
    return pl.pallas_call(
        _lstm_body,
        grid=(nblk,),
        in_specs=[
            pl.BlockSpec((EBLK, A), lambda i: (i + blk0, 0)),
            pl.BlockSpec((256, 128), lambda i: (0, 0)),
        ],
        out_specs=[
            pl.BlockSpec((EBLK, DROW), lambda i: (i, 0)),
            pl.BlockSpec((EBLK, DROW), lambda i: (i, 0)),
        ],
        out_shape=[
            jax.ShapeDtypeStruct((nblk * EBLK, DROW), jnp.float32),
            jax.ShapeDtypeStruct((nblk * EBLK, DROW), jnp.float32),
        ],
        interpret=interpret,
    )(edge_attr, W)


def _run_stage2(data_out, data_in, src, tgt, zeros_hbm, n_nodes):
    E = src.shape[0]
    NS = 16
    ET = E // NS
    CH = 128
    NFULL = ET // CH
    TAIL = ET - NFULL * CH
    TR = (n_nodes // NS) // 8 * 8
    TREM = n_nodes - TR * NS

    mesh = plsc.VectorSubcoreMesh(core_axis_name="c", subcore_axis_name="s")

    scratch = [
        pltpu.VMEM((CH, DROW), jnp.float32),
        pltpu.VMEM((CH,), jnp.int32),
        pltpu.VMEM_SHARED((n_nodes, DROW), jnp.float32),
    ]
    if TAIL:
        scratch += [
            pltpu.VMEM((TAIL, DROW), jnp.float32),
            pltpu.VMEM((TAIL,), jnp.int32),
        ]

    @functools.partial(
        pl.kernel,
        mesh=mesh,
        out_type=[
            jax.ShapeDtypeStruct((n_nodes, DROW), jnp.float32),
            jax.ShapeDtypeStruct((n_nodes, DROW), jnp.float32),
        ],
        scratch_types=scratch,
    )
    def sck(do_hbm, di_hbm, src_hbm, tgt_hbm, z_hbm, oo_hbm, oi_hbm,
            buf, idxv, acc, *tailbufs):
        c = lax.axis_index("c")
        s = lax.axis_index("s")
        pltpu.sync_copy(z_hbm.at[pl.ds(0, TR)], acc.at[pl.ds(s * TR, TR)])
        if TREM:
            @pl.when(s == NS - 1)
            def _():
                pltpu.sync_copy(z_hbm.at[pl.ds(0, TREM)],
                                acc.at[pl.ds(NS * TR, TREM)])
        plsc.subcore_barrier()

        def do_scatter(d_hbm, i_hbm):
            base = s * ET

            def body(k, carry):
                off = pl.multiple_of(base + k * CH, 8)
                pltpu.sync_copy(i_hbm.at[pl.ds(off, CH)], idxv)
                pltpu.sync_copy(d_hbm.at[pl.ds(off, CH)], buf)
                pltpu.sync_copy(buf, acc.at[idxv], add=True)
                return carry

            lax.fori_loop(0, NFULL, body, 0)
            if TAIL:
                tbuf, tidx = tailbufs
                off = base + NFULL * CH
                pltpu.sync_copy(i_hbm.at[pl.ds(off, TAIL)], tidx)
                pltpu.sync_copy(d_hbm.at[pl.ds(off, TAIL)], tbuf)
                pltpu.sync_copy(tbuf, acc.at[tidx], add=True)

        @pl.when(c == 0)
        def _():
            do_scatter(do_hbm, src_hbm)

        @pl.when(c == 1)
        def _():
            do_scatter(di_hbm, tgt_hbm)

        plsc.subcore_barrier()

        def writeout(o_hbm):
            pltpu.sync_copy(acc.at[pl.ds(s * TR, TR)], o_hbm.at[pl.ds(s * TR, TR)])
            if TREM:
                @pl.when(s == NS - 1)
                def _():
                    pltpu.sync_copy(acc.at[pl.ds(NS * TR, TREM)],
                                    o_hbm.at[pl.ds(NS * TR, TREM)])

        @pl.when(c == 0)
        def _():
            writeout(oo_hbm)

        @pl.when(c == 1)
        def _():
            writeout(oi_hbm)

    return sck(data_out, data_in, src, tgt, zeros_hbm)


def _mlp_body(x_ref, *refs):
    si_refs = refs[0:NHALF]
    so_refs = refs[NHALF:2 * NHALF]
    (p1x_ref, p1i_ref, p1o_ref, pb1_ref,
     p2_ref, pb2_ref, p3_ref, pb3_ref, p4_ref, pb4_ref,
     v1x_ref, v1i_ref, v1o_ref, vb1_ref, v2_ref, vb2_ref,
     v_ref, pi_ref) = refs[2 * NHALF:]
    f32 = jnp.float32
    x = x_ref[...]
    si = si_refs[0][...]
    so = so_refs[0][...]
    for r in si_refs[1:]:
        si = si + r[...]
    for r in so_refs[1:]:
        so = so + r[...]
    R = x.shape[0]
    cnt_i = jnp.maximum(si[:, CNT_COL:CNT_COL + 1], 1.0)
    cnt_o = jnp.maximum(so[:, CNT_COL:CNT_COL + 1], 1.0)
    inf = si / cnt_i
    outf = so / cnt_o

    def dot(a, b):
        return jnp.dot(a, b, preferred_element_type=f32)

    pi1 = jnp.maximum(dot(x, p1x_ref[...]) + dot(inf, p1i_ref[...])
                      + dot(outf, p1o_ref[...]) + pb1_ref[...], 0.0)
    pi2 = jnp.maximum(dot(pi1, p2_ref[...]) + pb2_ref[...], 0.0)
    pi3 = jnp.maximum(dot(pi2, p3_ref[...]) + pb3_ref[...], 0.0)
    pi4 = dot(pi3, p4_ref[...]) + pb4_ref[...]
    pi_ref[...] = pi4

    rows_per_graph = R // GBLK
    rg = lax.broadcasted_iota(jnp.int32, (GBLK, R), 1) // rows_per_graph
    gi = lax.broadcasted_iota(jnp.int32, (GBLK, R), 0)
    pool = (rg == gi).astype(f32) * (1.0 / rows_per_graph)
    mx = dot(pool, x)
    mi = dot(pool, inf)
    mo = dot(pool, outf)
    sv = jnp.maximum(dot(mx, v1x_ref[...]) + dot(mi, v1i_ref[...])
                     + dot(mo, v1o_ref[...]) + vb1_ref[...], 0.0)
    v_ref[...] = dot(sv, v2_ref[...]) + vb2_ref[...]


def _run_stage3(x, sums_in_list, sums_out_list, wts, n_graphs, interpret=False):
    N, XD = x.shape
    R = (N // n_graphs) * GBLK
    grid = (N // R,)

    def const_spec(a):
        return pl.BlockSpec(a.shape, lambda i: tuple(0 for _ in a.shape))

    in_specs = [
        pl.BlockSpec((R, XD), lambda i: (i, 0)),
    ] + [pl.BlockSpec((R, DROW), lambda i: (i, 0))] * (2 * NHALF) \
      + [const_spec(w) for w in wts]
    return pl.pallas_call(
        _mlp_body,
        grid=grid,
        in_specs=in_specs,
        out_specs=[
            pl.BlockSpec((GBLK, 1), lambda i: (i, 0)),
            pl.BlockSpec((R, 1), lambda i: (i, 0)),
        ],
        out_shape=[
            jax.ShapeDtypeStruct((n_graphs, 1), jnp.float32),
            jax.ShapeDtypeStruct((N, 1), jnp.float32),
        ],
        interpret=interpret,
    )(x, *sums_in_list, *sums_out_list, *wts)


def _nx_body(pir_ref, nx_ref):
    p = pir_ref[...]
    Gb = p.shape[0]
    neg = jnp.full((Gb, ACTION_SIZE - p.shape[1]), -999.0, jnp.float32)
    cat = jnp.concatenate([p, neg], axis=1)
    m = jnp.max(cat, axis=1, keepdims=True)
    lse = m + jnp.log(jnp.sum(jnp.exp(cat - m), axis=1, keepdims=True))
    nx_ref[...] = cat - lse


def _run_stage3b(pir, interpret=False):
    G, PG = pir.shape
    return pl.pallas_call(
        _nx_body,
        grid=(1,),
        in_specs=[pl.BlockSpec((G, PG), lambda i: (0, 0))],
        out_specs=pl.BlockSpec((G, ACTION_SIZE), lambda i: (0, 0)),
        out_shape=jax.ShapeDtypeStruct((G, ACTION_SIZE), jnp.float32),
        interpret=interpret,
    )(pir)


def _stage1_weights(embed_table, w_ih_f, w_hh_f, b_ih_f, b_hh_f,
                    w_ih_b, w_hh_b, b_ih_b, b_hh_b):
    f32 = jnp.float32
    EMBD = embed_table.shape[1]
    embp = jnp.zeros((VOCAB_PAD, EMBD), f32).at[:embed_table.shape[0]].set(embed_table)
    W = jnp.zeros((256, 128), f32)
    W = W.at[0:128, 0:32].set((embp @ w_ih_f.T).T)
    W = W.at[0:128, 32:64].set(w_hh_f)
    W = W.at[128:256, 64:96].set((embp @ w_ih_b.T).T)
    W = W.at[128:256, 96:128].set(w_hh_b)
    bias = jnp.concatenate([b_ih_f + b_hh_f, b_ih_b + b_hh_b])
    W = W.at[:, VOCAB_PAD - 1].set(bias)
    gate_row = jnp.arange(256) % 128
    is_sig = (gate_row < 64) | (gate_row >= 96)
    sc = jnp.where(is_sig, 0.5, 1.0).astype(f32)
    return W * sc[:, None]

def kernel(x, edge_index, edge_attr, batch, embed_table,
           w_ih_f, w_hh_f, b_ih_f, b_hh_f, w_ih_b, w_hh_b, b_ih_b, b_hh_b,
           pw1, pb1, pw2, pb2, pw3, pb3, pw4, pb4, vw1, vb1, vw2, vb2):
    f32 = jnp.float32
    N, XD = x.shape
    E = edge_index.shape[1]
    G = 200
    PG = N // G

    W = _stage1_weights(embed_table, w_ih_f, w_hh_f, b_ih_f, b_hh_f,
                        w_ih_b, w_hh_b, b_ih_b, b_hh_b)
    zeros_hbm = jnp.zeros((N // 16, DROW), f32)
    nb = E // EBLK
    bounds = [0]
    for h in range(NHALF):
        bounds.append(bounds[-1] + (nb // NHALF + (1 if h < nb % NHALF else 0)) * EBLK)
    sums_out_list, sums_in_list = [], []
    for h in range(NHALF):
        sl = slice(bounds[h], bounds[h + 1])
        data_out, data_in = _run_stage1(edge_attr, W, blk0=bounds[h] // EBLK,
                                        nblk=(bounds[h + 1] - bounds[h]) // EBLK)
        so_h, si_h = _run_stage2(data_out, data_in, edge_index[0, sl],
                                 edge_index[1, sl], zeros_hbm, N)
        sums_out_list.append(so_h)
        sums_in_list.append(si_h)

    D2 = SND + 2 * H
    p1t = pw1.T
    P1x = p1t[0:XD]
    P1i = jnp.zeros((DROW, p1t.shape[1]), f32).at[0:D2].set(p1t[XD:XD + D2])
    P1o = jnp.zeros((DROW, p1t.shape[1]), f32).at[0:D2].set(p1t[XD + D2:])
    v1t = vw1.T
    V1x = v1t[0:XD]
    V1i = jnp.zeros((DROW, v1t.shape[1]), f32).at[0:D2].set(v1t[XD:XD + D2])
    V1o = jnp.zeros((DROW, v1t.shape[1]), f32).at[0:D2].set(v1t[XD + D2:])
    wts = [
        P1x, P1i, P1o, pb1.reshape(1, -1),
        pw2.T, pb2.reshape(1, -1), pw3.T, pb3.reshape(1, -1),
        pw4.T, pb4.reshape(1, -1),
        V1x, V1i, V1o, vb1.reshape(1, -1), vw2.T, vb2.reshape(1, -1),
    ]
    v, pi = _run_stage3(x, sums_in_list, sums_out_list, wts, G)

    new_x = _run_stage3b(pi.reshape(G, PG))
    return new_x, v
